# baseline (speedup 1.0000x reference)
import functools

import jax
import jax.numpy as jnp
from jax import lax
from jax.experimental import pallas as pl
from jax.experimental.pallas import tpu as pltpu
from jax.experimental.pallas import tpu_sc as plsc

HCH = (0, 1, 2, 3, 4, 6)
MPOS = (0, 0, 0, 0, 1, 2, 0, 2, 4, 0, 3, 5)

NC = 2
NS = 16
L = 16
NW = NC * NS
H = W = 512
ROWS_PER_W = H // NW
PIX_PER_W = ROWS_PER_W * W
GROUPS = PIX_PER_W // L
GPR = W // L
UNROLL = 16
NU = 2 * len(HCH)

_mesh = plsc.VectorSubcoreMesh(core_axis_name="c", subcore_axis_name="s")
_cparams = pltpu.CompilerParams(needs_layout_passes=False)


@functools.partial(
    pl.kernel,
    out_type=jax.ShapeDtypeStruct((NW, NU * 256), jnp.float32),
    scratch_types=[
        pltpu.VMEM((NU * 256,), jnp.float32),
        pltpu.VMEM((ROWS_PER_W, W), jnp.float32),
        pltpu.VMEM((ROWS_PER_W, W), jnp.float32),
        pltpu.VMEM((ROWS_PER_W, W), jnp.float32),
        pltpu.VMEM((ROWS_PER_W, W), jnp.float32),
        pltpu.SemaphoreType.DMA,
        pltpu.SemaphoreType.DMA,
        pltpu.SemaphoreType.DMA,
        pltpu.SemaphoreType.DMA,
    ],
    mesh=_mesh,
    compiler_params=_cparams,
)
def _hist_sc(dstp, refp, parts, histv, pix0, pix1, pix2, pix3,
             sem0, sem1, sem2, sem3):
    wid = lax.axis_index("s") * NC + lax.axis_index("c")
    rbase = wid * ROWS_PER_W
    ones = jnp.ones((L,), jnp.float32)
    pixbufs = (pix0, pix1, pix2, pix3)
    sems = (sem0, sem1, sem2, sem3)

    @plsc.parallel_loop(0, (NU * 256) // L, unroll=UNROLL)
    def zero_body(i):
        histv[pl.ds(i * L, L)] = jnp.zeros((L,), jnp.float32)

    def src_slice(u):
        src = dstp if u < len(HCH) else refp
        b, c = divmod(HCH[u % len(HCH)], 3)
        return src.at[b, c, pl.ds(rbase, ROWS_PER_W), :]

    cps = [None] * NU
    for u in range(3):
        cps[u] = pltpu.async_copy(src_slice(u), pixbufs[u % 4], sems[u % 4])
    for u in range(NU):
        cps[u].wait()
        pixv = pixbufs[u % 4]
        ubase = u * 256

        @plsc.parallel_loop(0, GROUPS, unroll=UNROLL)
        def hist_body(i):
            r = i // GPR
            col = (i % GPR) * L
            v = pixv[r, pl.ds(col, L)]
            q = jnp.minimum(jnp.maximum(v * 256.0, 0.0), 255.0)
            idx = q.astype(jnp.int32) + ubase
            plsc.addupdate_scatter(histv, [idx], ones)

        if u + 3 < NU:
            cps[u + 3] = pltpu.async_copy(
                src_slice(u + 3), pixbufs[(u + 3) % 4], sems[(u + 3) % 4])

    pltpu.sync_copy(histv, parts.at[wid])


def _table_body(parts_ref, lut_ref):
    parts = parts_ref[...]
    h = jnp.sum(parts, axis=0)
    hd = jnp.stack([h[u * 256:(u + 1) * 256] for u in range(len(HCH))])
    hr = jnp.stack([h[(len(HCH) + u) * 256:(len(HCH) + u + 1) * 256]
                    for u in range(len(HCH))])
    tri = (lax.broadcasted_iota(jnp.int32, (256, 256), 0)
           <= lax.broadcasted_iota(jnp.int32, (256, 256), 1)
           ).astype(jnp.float32)
    cd = jnp.dot(hd, tri, preferred_element_type=jnp.float32)
    cr = jnp.dot(hr, tri, preferred_element_type=jnp.float32)
    g = (cd[:, :, None] - cr[:, None, :] >= 0.0).astype(jnp.float32)
    tab = jnp.sum(g, axis=2) - 1.0
    tab = jnp.minimum(jnp.maximum(tab, 0.0), 255.0) * (1.0 / 255.0)
    lut_ref[...] = tab


def _table_tc(parts):
    return pl.pallas_call(
        _table_body,
        out_shape=jax.ShapeDtypeStruct((len(HCH), 256), jnp.float32),
    )(parts)


@functools.partial(
    pl.kernel,
    out_type=jax.ShapeDtypeStruct((4, 3, H, W), jnp.float32),
    scratch_types=[
        pltpu.VMEM((len(HCH) * 256,), jnp.float32),
        pltpu.VMEM((ROWS_PER_W, W), jnp.float32),
        pltpu.VMEM((ROWS_PER_W, W), jnp.float32),
        pltpu.VMEM((ROWS_PER_W, W), jnp.float32),
        pltpu.VMEM((ROWS_PER_W, W), jnp.float32),
        pltpu.VMEM((ROWS_PER_W, W), jnp.float32),
        pltpu.VMEM((ROWS_PER_W, W), jnp.float32),
        pltpu.VMEM((ROWS_PER_W, W), jnp.float32),
        pltpu.VMEM((ROWS_PER_W, W), jnp.float32),
        pltpu.SemaphoreType.DMA,
        pltpu.SemaphoreType.DMA,
        pltpu.SemaphoreType.DMA,
        pltpu.SemaphoreType.DMA,
        pltpu.SemaphoreType.DMA,
        pltpu.SemaphoreType.DMA,
        pltpu.SemaphoreType.DMA,
        pltpu.SemaphoreType.DMA,
    ],
    mesh=_mesh,
    compiler_params=_cparams,
)
def _gather_sc(dstp, lutp, outp, lutv,
               pix0, pix1, pix2, pix3, out0, out1, out2, out3,
               semi0, semi1, semi2, semi3, semo0, semo1, semo2, semo3):
    wid = lax.axis_index("s") * NC + lax.axis_index("c")
    rbase = wid * ROWS_PER_W
    pixbufs = (pix0, pix1, pix2, pix3)
    outbufs = (out0, out1, out2, out3)
    isems = (semi0, semi1, semi2, semi3)
    osems = (semo0, semo1, semo2, semo3)
    pltpu.sync_copy(lutp, lutv)

    def in_slice(ch):
        b, c = divmod(ch, 3)
        return dstp.at[b, c, pl.ds(rbase, ROWS_PER_W), :]

    cpi = [None] * 12
    cpo = [None] * 12
    for ch in range(3):
        cpi[ch] = pltpu.async_copy(in_slice(ch), pixbufs[ch % 4], isems[ch % 4])
    for ch in range(12):
        cpi[ch].wait()
        if ch >= 4:
            cpo[ch - 4].wait()
        pixv = pixbufs[ch % 4]
        outv = outbufs[ch % 4]
        cbase = MPOS[ch] * 256

        @plsc.parallel_loop(0, GROUPS, unroll=UNROLL)
        def body(i):
            r = i // GPR
            col = (i % GPR) * L
            v = pixv[r, pl.ds(col, L)]
            t = jnp.minimum(jnp.maximum(v * 255.0, 0.0), 255.0)
            idx = t.astype(jnp.int32) + cbase
            outv[r, pl.ds(col, L)] = plsc.load_gather(lutv, [idx])

        if ch + 3 < 12:
            cpi[ch + 3] = pltpu.async_copy(
                in_slice(ch + 3), pixbufs[(ch + 3) % 4], isems[(ch + 3) % 4])
        b, c = divmod(ch, 3)
        cpo[ch] = pltpu.async_copy(
            outv, outp.at[b, c, pl.ds(rbase, ROWS_PER_W), :], osems[ch % 4])
    for ch in range(8, 12):
        cpo[ch].wait()


def kernel(dst, ref):
    parts = _hist_sc(dst, ref)
    lut = _table_tc(parts)
    return _gather_sc(dst, lut.reshape(len(HCH) * 256))


# speedup vs baseline: 1.0685x; 1.0685x over previous
import functools

import jax
import jax.numpy as jnp
from jax import lax
from jax.experimental import pallas as pl
from jax.experimental.pallas import tpu as pltpu
from jax.experimental.pallas import tpu_sc as plsc

HCH = (0, 1, 2, 3, 4, 6)
MPOS = (0, 0, 0, 0, 1, 2, 0, 2, 4, 0, 3, 5)

NC = 2
NS = 16
L = 16
NW = NC * NS
H = W = 512
ROWS_PER_W = H // NW
PIX_PER_W = ROWS_PER_W * W
GROUPS = PIX_PER_W // L
GPR = W // L
UNROLL = 8
NU = 2 * len(HCH)

_mesh = plsc.VectorSubcoreMesh(core_axis_name="c", subcore_axis_name="s")
_cparams = pltpu.CompilerParams(needs_layout_passes=False)


@functools.partial(
    pl.kernel,
    out_type=jax.ShapeDtypeStruct((NW, NU * 256), jnp.float32),
    scratch_types=[
        pltpu.VMEM((NU * 256,), jnp.float32),
        pltpu.VMEM((ROWS_PER_W, W), jnp.float32),
        pltpu.VMEM((ROWS_PER_W, W), jnp.float32),
        pltpu.VMEM((ROWS_PER_W, W), jnp.float32),
        pltpu.VMEM((ROWS_PER_W, W), jnp.float32),
        pltpu.SemaphoreType.DMA,
        pltpu.SemaphoreType.DMA,
        pltpu.SemaphoreType.DMA,
        pltpu.SemaphoreType.DMA,
    ],
    mesh=_mesh,
    compiler_params=_cparams,
)
def _hist_sc(dstp, refp, parts, histv, pix0, pix1, pix2, pix3,
             sem0, sem1, sem2, sem3):
    wid = lax.axis_index("s") * NC + lax.axis_index("c")
    rbase = wid * ROWS_PER_W
    ones = jnp.ones((L,), jnp.float32)
    pixbufs = (pix0, pix1, pix2, pix3)
    sems = (sem0, sem1, sem2, sem3)

    @plsc.parallel_loop(0, (NU * 256) // L, unroll=UNROLL)
    def zero_body(i):
        histv[pl.ds(i * L, L)] = jnp.zeros((L,), jnp.float32)

    def src_slice(u):
        src = dstp if u < len(HCH) else refp
        b, c = divmod(HCH[u % len(HCH)], 3)
        return src.at[b, c, pl.ds(rbase, ROWS_PER_W), :]

    cps = [None] * NU
    for u in range(3):
        cps[u] = pltpu.async_copy(src_slice(u), pixbufs[u % 4], sems[u % 4])
    for u in range(NU):
        cps[u].wait()
        pixv = pixbufs[u % 4]
        ubase = u * 256

        @plsc.parallel_loop(0, GROUPS, unroll=UNROLL)
        def hist_body(i):
            r = i // GPR
            col = (i % GPR) * L
            v = pixv[r, pl.ds(col, L)]
            q = jnp.minimum(jnp.maximum(v * 256.0, 0.0), 255.0)
            idx = q.astype(jnp.int32) + ubase
            plsc.addupdate_scatter(histv, [idx], ones)

        if u + 3 < NU:
            cps[u + 3] = pltpu.async_copy(
                src_slice(u + 3), pixbufs[(u + 3) % 4], sems[(u + 3) % 4])

    pltpu.sync_copy(histv, parts.at[wid])


def _table_body(parts_ref, lut_ref):
    parts = parts_ref[...]
    h = jnp.sum(parts, axis=0)
    hd = jnp.stack([h[u * 256:(u + 1) * 256] for u in range(len(HCH))])
    hr = jnp.stack([h[(len(HCH) + u) * 256:(len(HCH) + u + 1) * 256]
                    for u in range(len(HCH))])
    tri = (lax.broadcasted_iota(jnp.int32, (256, 256), 0)
           <= lax.broadcasted_iota(jnp.int32, (256, 256), 1)
           ).astype(jnp.float32)
    cd = jnp.dot(hd, tri, preferred_element_type=jnp.float32)
    cr = jnp.dot(hr, tri, preferred_element_type=jnp.float32)
    g = (cd[:, :, None] - cr[:, None, :] >= 0.0).astype(jnp.float32)
    tab = jnp.sum(g, axis=2) - 1.0
    tab = jnp.minimum(jnp.maximum(tab, 0.0), 255.0) * (1.0 / 255.0)
    lut_ref[...] = tab


def _table_tc(parts):
    return pl.pallas_call(
        _table_body,
        out_shape=jax.ShapeDtypeStruct((len(HCH), 256), jnp.float32),
    )(parts)


@functools.partial(
    pl.kernel,
    out_type=jax.ShapeDtypeStruct((4, 3, H, W), jnp.float32),
    scratch_types=[
        pltpu.VMEM((len(HCH) * 256,), jnp.float32),
        pltpu.VMEM((ROWS_PER_W, W), jnp.float32),
        pltpu.VMEM((ROWS_PER_W, W), jnp.float32),
        pltpu.VMEM((ROWS_PER_W, W), jnp.float32),
        pltpu.VMEM((ROWS_PER_W, W), jnp.float32),
        pltpu.VMEM((ROWS_PER_W, W), jnp.float32),
        pltpu.VMEM((ROWS_PER_W, W), jnp.float32),
        pltpu.VMEM((ROWS_PER_W, W), jnp.float32),
        pltpu.VMEM((ROWS_PER_W, W), jnp.float32),
        pltpu.SemaphoreType.DMA,
        pltpu.SemaphoreType.DMA,
        pltpu.SemaphoreType.DMA,
        pltpu.SemaphoreType.DMA,
        pltpu.SemaphoreType.DMA,
        pltpu.SemaphoreType.DMA,
        pltpu.SemaphoreType.DMA,
        pltpu.SemaphoreType.DMA,
    ],
    mesh=_mesh,
    compiler_params=_cparams,
)
def _gather_sc(dstp, lutp, outp, lutv,
               pix0, pix1, pix2, pix3, out0, out1, out2, out3,
               semi0, semi1, semi2, semi3, semo0, semo1, semo2, semo3):
    wid = lax.axis_index("s") * NC + lax.axis_index("c")
    rbase = wid * ROWS_PER_W
    pixbufs = (pix0, pix1, pix2, pix3)
    outbufs = (out0, out1, out2, out3)
    isems = (semi0, semi1, semi2, semi3)
    osems = (semo0, semo1, semo2, semo3)
    pltpu.sync_copy(lutp, lutv)

    def in_slice(ch):
        b, c = divmod(ch, 3)
        return dstp.at[b, c, pl.ds(rbase, ROWS_PER_W), :]

    cpi = [None] * 12
    cpo = [None] * 12
    for ch in range(3):
        cpi[ch] = pltpu.async_copy(in_slice(ch), pixbufs[ch % 4], isems[ch % 4])
    for ch in range(12):
        cpi[ch].wait()
        if ch >= 4:
            cpo[ch - 4].wait()
        pixv = pixbufs[ch % 4]
        outv = outbufs[ch % 4]
        cbase = MPOS[ch] * 256

        @plsc.parallel_loop(0, GROUPS, unroll=UNROLL)
        def body(i):
            r = i // GPR
            col = (i % GPR) * L
            v = pixv[r, pl.ds(col, L)]
            t = jnp.minimum(jnp.maximum(v * 255.0, 0.0), 255.0)
            idx = t.astype(jnp.int32) + cbase
            outv[r, pl.ds(col, L)] = plsc.load_gather(lutv, [idx])

        if ch + 3 < 12:
            cpi[ch + 3] = pltpu.async_copy(
                in_slice(ch + 3), pixbufs[(ch + 3) % 4], isems[(ch + 3) % 4])
        b, c = divmod(ch, 3)
        cpo[ch] = pltpu.async_copy(
            outv, outp.at[b, c, pl.ds(rbase, ROWS_PER_W), :], osems[ch % 4])
    for ch in range(8, 12):
        cpo[ch].wait()


def kernel(dst, ref):
    parts = _hist_sc(dst, ref)
    lut = _table_tc(parts)
    return _gather_sc(dst, lut.reshape(len(HCH) * 256))
